# Initial kernel scaffold; baseline (speedup 1.0000x reference)
import jax
import jax.numpy as jnp
from jax.experimental import pallas as pl


def kernel(weight):
    raise NotImplementedError("write your pallas kernel here")


# speedup vs baseline: 26.8754x; 26.8754x over previous
import functools

import jax
import jax.numpy as jnp
from jax import lax
from jax.experimental import pallas as pl
from jax.experimental.pallas import tpu as pltpu
from jax.experimental.pallas import tpu_sc as plsc

_SPARSITY = 0.5
_TEMP = 1e-05

_N = 4096 * 4096
_LIM = int(min(max(int((1.0 - _SPARSITY) * _N), 0), _N - 2))
_R_HI = _N - 1 - _LIM
_R_LO = _N - 2 - _LIM

_NTILES = 32
_PER_TILE = _N // _NTILES
_CHUNK = 4096
_NCHUNK = _PER_TILE // _CHUNK
_VPC = _CHUNK // 16
_B1 = 4096
_B2 = 2048

_mesh = plsc.VectorSubcoreMesh(core_axis_name="c", subcore_axis_name="s")
_sc_params = pltpu.CompilerParams(needs_layout_passes=False)


def _wid():
    return lax.axis_index("s") * 2 + lax.axis_index("c")


@functools.partial(
    pl.kernel,
    out_type=jax.ShapeDtypeStruct((_NTILES, _B1), jnp.int32),
    mesh=_mesh,
    compiler_params=_sc_params,
    scratch_types=[
        pltpu.VMEM((_CHUNK,), jnp.int32),
        pltpu.VMEM((16 * _B1,), jnp.int32),
    ],
)
def _hist1(w_hbm, out_hbm, buf, hist):
    wid = _wid()
    base = wid * _PER_TILE
    iota = lax.iota(jnp.int32, 16)
    ones = jnp.ones((16,), jnp.int32)
    zeros = jnp.zeros((16,), jnp.int32)

    def zbody(i, c):
        hist[pl.ds(i * 16, 16)] = zeros
        return c

    lax.fori_loop(0, (16 * _B1) // 16, zbody, 0)

    def cbody(c, carry):
        pltpu.sync_copy(w_hbm.at[pl.ds(base + c * _CHUNK, _CHUNK)], buf)

        def vbody(i, cc):
            q = buf[pl.ds(i * 16, 16)] & jnp.int32(0x7FFFFFFF)
            b = q >> 19
            plsc.addupdate_scatter(hist, [iota * _B1 + b], ones)
            return cc

        return lax.fori_loop(0, _VPC, vbody, carry)

    lax.fori_loop(0, _NCHUNK, cbody, 0)

    def rbody(j, c):
        acc = hist[pl.ds(j * 16, 16)]
        for k in range(1, 16):
            acc = acc + hist[pl.ds(k * _B1 + j * 16, 16)]
        hist[pl.ds(j * 16, 16)] = acc
        return c

    lax.fori_loop(0, _B1 // 16, rbody, 0)
    pltpu.sync_copy(hist.at[pl.ds(0, _B1)], out_hbm.at[wid])


@functools.partial(
    pl.kernel,
    out_type=jax.ShapeDtypeStruct((_NTILES, 2 * _B2), jnp.int32),
    mesh=_mesh,
    compiler_params=_sc_params,
    scratch_types=[
        pltpu.VMEM((_CHUNK,), jnp.int32),
        pltpu.VMEM((32,), jnp.int32),
        pltpu.VMEM((32 * _B2,), jnp.int32),
    ],
)
def _hist2(w_hbm, targets_hbm, out_hbm, buf, tvec, hist):
    wid = _wid()
    base = wid * _PER_TILE
    iota = lax.iota(jnp.int32, 16)
    ones = jnp.ones((16,), jnp.int32)
    zeros = jnp.zeros((16,), jnp.int32)

    pltpu.sync_copy(targets_hbm, tvec)
    pa = tvec[pl.ds(0, 16)]
    pb = tvec[pl.ds(16, 16)]

    def zbody(i, c):
        hist[pl.ds(i * 16, 16)] = zeros
        return c

    lax.fori_loop(0, (32 * _B2) // 16, zbody, 0)

    def cbody(c, carry):
        pltpu.sync_copy(w_hbm.at[pl.ds(base + c * _CHUNK, _CHUNK)], buf)

        def vbody(i, cc):
            q = buf[pl.ds(i * 16, 16)] & jnp.int32(0x7FFFFFFF)
            pfx = q >> 19
            idx = iota * _B2 + ((q >> 8) & (_B2 - 1))
            plsc.addupdate_scatter(hist, [idx], ones, mask=pfx == pa)
            plsc.addupdate_scatter(hist, [idx + 16 * _B2], ones, mask=pfx == pb)
            return cc

        return lax.fori_loop(0, _VPC, vbody, carry)

    lax.fori_loop(0, _NCHUNK, cbody, 0)

    for r in range(2):
        def rbody(j, c, r=r):
            acc = hist[pl.ds(r * 16 * _B2 + j * 16, 16)]
            for k in range(1, 16):
                acc = acc + hist[pl.ds(r * 16 * _B2 + k * _B2 + j * 16, 16)]
            hist[pl.ds(r * _B2 + j * 16, 16)] = acc
            return c

        lax.fori_loop(0, _B2 // 16, rbody, 0)
    pltpu.sync_copy(hist.at[pl.ds(0, 2 * _B2)], out_hbm.at[wid])


def _mask_body(t2_ref, w_ref, o_ref):
    w = w_ref[...]
    d = (t2_ref[0, 0] - w * w) * jnp.float32(1.0 / _TEMP)
    o_ref[...] = w / (1.0 + jnp.exp(d))


_mask = pl.pallas_call(
    _mask_body,
    grid=(16,),
    in_specs=[
        pl.BlockSpec((1, 1), lambda i: (0, 0)),
        pl.BlockSpec((256, 4096), lambda i: (i, 0)),
    ],
    out_specs=pl.BlockSpec((256, 4096), lambda i: (i, 0)),
    out_shape=jax.ShapeDtypeStruct((4096, 4096), jnp.float32),
)


def kernel(weight):
    wflat = lax.bitcast_convert_type(weight, jnp.int32).reshape(-1)

    h1 = jnp.sum(_hist1(wflat), axis=0)
    c1 = jnp.cumsum(h1)
    excl1 = c1 - h1
    b_hi = jnp.searchsorted(c1, _R_HI, side="right").astype(jnp.int32)
    b_lo = jnp.searchsorted(c1, _R_LO, side="right").astype(jnp.int32)
    r_hi = jnp.int32(_R_HI) - excl1[b_hi]
    r_lo = jnp.int32(_R_LO) - excl1[b_lo]

    targets = jnp.concatenate(
        [jnp.full((16,), b_hi, jnp.int32), jnp.full((16,), b_lo, jnp.int32)]
    )
    h2 = jnp.sum(_hist2(wflat, targets), axis=0)
    ha, hb = h2[:_B2], h2[_B2:]
    m_hi = jnp.searchsorted(jnp.cumsum(ha), r_hi, side="right").astype(jnp.int32)
    m_lo = jnp.searchsorted(jnp.cumsum(hb), r_lo, side="right").astype(jnp.int32)

    q_hi = (b_hi << 19) | (m_hi << 8) | 128
    q_lo = (b_lo << 19) | (m_lo << 8) | 128
    wh = lax.bitcast_convert_type(q_hi, jnp.float32)
    wt = lax.bitcast_convert_type(q_lo, jnp.float32)
    t = 0.5 * (wh + wt)
    t2 = (t * t).reshape(1, 1)

    return _mask(t2, weight)


# speedup vs baseline: 36.3503x; 1.3525x over previous
import functools

import jax
import jax.numpy as jnp
from jax import lax
from jax.experimental import pallas as pl
from jax.experimental.pallas import tpu as pltpu
from jax.experimental.pallas import tpu_sc as plsc

_SPARSITY = 0.5
_TEMP = 1e-05

_N = 4096 * 4096
_LIM = int(min(max(int((1.0 - _SPARSITY) * _N), 0), _N - 2))
_R_HI = _N - 1 - _LIM
_R_LO = _N - 2 - _LIM

_NTILES = 32
_PER_TILE = _N // _NTILES
_CHUNK = 8192
_NCHUNK = _PER_TILE // _CHUNK
_NPAIR = _NCHUNK // 2
_UNROLL = 8
_B1 = 4096
_B2 = 2048

_mesh = plsc.VectorSubcoreMesh(core_axis_name="c", subcore_axis_name="s")
_sc_params = pltpu.CompilerParams(needs_layout_passes=False)


def _wid():
    return lax.axis_index("s") * 2 + lax.axis_index("c")


def _zero(hist, nwords):
    zeros = jnp.zeros((16,), jnp.int32)

    def zbody(i, c):
        for j in range(_UNROLL):
            hist[pl.ds(i * 16 * _UNROLL + j * 16, 16)] = zeros
        return c

    lax.fori_loop(0, nwords // (16 * _UNROLL), zbody, 0)


def _stream_chunks(w_hbm, base, bufa, bufb, sema, semb, process):
    def src(c):
        return w_hbm.at[pl.ds(base + c * _CHUNK, _CHUNK)]

    pltpu.async_copy(src(0), bufa, sema)

    def pair_body(p, carry):
        c = 2 * p
        pltpu.async_copy(src(c + 1), bufb, semb)
        pltpu.make_async_copy(src(0), bufa, sema).wait()
        process(bufa)
        nxt = jnp.minimum(c + 2, _NCHUNK - 2)
        pltpu.async_copy(src(nxt), bufa, sema)
        pltpu.make_async_copy(src(0), bufb, semb).wait()
        process(bufb)
        return carry

    lax.fori_loop(0, _NPAIR, pair_body, 0)
    pltpu.make_async_copy(src(0), bufa, sema).wait()


@functools.partial(
    pl.kernel,
    out_type=jax.ShapeDtypeStruct((_NTILES, _B1), jnp.int32),
    mesh=_mesh,
    compiler_params=_sc_params,
    scratch_types=[
        pltpu.VMEM((_CHUNK,), jnp.int32),
        pltpu.VMEM((_CHUNK,), jnp.int32),
        pltpu.VMEM((16 * _B1,), jnp.int32),
        pltpu.SemaphoreType.DMA,
        pltpu.SemaphoreType.DMA,
    ],
)
def _hist1(w_hbm, out_hbm, bufa, bufb, hist, sema, semb):
    wid = _wid()
    base = wid * _PER_TILE
    lane_off = lax.iota(jnp.int32, 16) * _B1
    ones = jnp.ones((16,), jnp.int32)

    _zero(hist, 16 * _B1)

    def process(buf):
        def vbody(i, cc):
            for j in range(_UNROLL):
                q = buf[pl.ds(i * 16 * _UNROLL + j * 16, 16)] & jnp.int32(0x7FFFFFFF)
                plsc.addupdate_scatter(hist, [lane_off + (q >> 19)], ones)
            return cc

        lax.fori_loop(0, _CHUNK // (16 * _UNROLL), vbody, 0)

    _stream_chunks(w_hbm, base, bufa, bufb, sema, semb, process)

    def rbody(j, c):
        acc = hist[pl.ds(j * 16, 16)]
        for k in range(1, 16):
            acc = acc + hist[pl.ds(k * _B1 + j * 16, 16)]
        hist[pl.ds(j * 16, 16)] = acc
        return c

    lax.fori_loop(0, _B1 // 16, rbody, 0)
    pltpu.sync_copy(hist.at[pl.ds(0, _B1)], out_hbm.at[wid])


@functools.partial(
    pl.kernel,
    out_type=jax.ShapeDtypeStruct((_NTILES, 2 * _B2), jnp.int32),
    mesh=_mesh,
    compiler_params=_sc_params,
    scratch_types=[
        pltpu.VMEM((_CHUNK,), jnp.int32),
        pltpu.VMEM((_CHUNK,), jnp.int32),
        pltpu.VMEM((32,), jnp.int32),
        pltpu.VMEM((32 * _B2,), jnp.int32),
        pltpu.SemaphoreType.DMA,
        pltpu.SemaphoreType.DMA,
    ],
)
def _hist2(w_hbm, targets_hbm, out_hbm, bufa, bufb, tvec, hist, sema, semb):
    wid = _wid()
    base = wid * _PER_TILE
    lane_off = lax.iota(jnp.int32, 16) * _B2
    ones = jnp.ones((16,), jnp.int32)

    pltpu.sync_copy(targets_hbm, tvec)
    pa = tvec[pl.ds(0, 16)]
    pb = tvec[pl.ds(16, 16)]
    b_off = jnp.where(pa != pb, jnp.int32(16 * _B2), jnp.int32(0))

    _zero(hist, 32 * _B2)

    def process(buf):
        def vbody(i, cc):
            for j in range(_UNROLL):
                q = buf[pl.ds(i * 16 * _UNROLL + j * 16, 16)] & jnp.int32(0x7FFFFFFF)
                pfx = q >> 19
                is_b = pfx == pb
                idx = lane_off + ((q >> 8) & (_B2 - 1)) + jnp.where(is_b, b_off, 0)
                plsc.addupdate_scatter(
                    hist, [idx], ones, mask=(pfx == pa) | is_b
                )
            return cc

        lax.fori_loop(0, _CHUNK // (16 * _UNROLL), vbody, 0)

    _stream_chunks(w_hbm, base, bufa, bufb, sema, semb, process)

    for r in range(2):
        def rbody(j, c, r=r):
            acc = hist[pl.ds(r * 16 * _B2 + j * 16, 16)]
            for k in range(1, 16):
                acc = acc + hist[pl.ds(r * 16 * _B2 + k * _B2 + j * 16, 16)]
            hist[pl.ds(r * _B2 + j * 16, 16)] = acc
            return c

        lax.fori_loop(0, _B2 // 16, rbody, 0)
    pltpu.sync_copy(hist.at[pl.ds(0, 2 * _B2)], out_hbm.at[wid])


def _mask_body(t2_ref, w_ref, o_ref):
    w = w_ref[...]
    d = (t2_ref[0, 0] - w * w) * jnp.float32(1.0 / _TEMP)
    o_ref[...] = w / (1.0 + jnp.exp(d))


_mask = pl.pallas_call(
    _mask_body,
    grid=(16,),
    in_specs=[
        pl.BlockSpec((1, 1), lambda i: (0, 0)),
        pl.BlockSpec((256, 4096), lambda i: (i, 0)),
    ],
    out_specs=pl.BlockSpec((256, 4096), lambda i: (i, 0)),
    out_shape=jax.ShapeDtypeStruct((4096, 4096), jnp.float32),
)


def kernel(weight):
    wflat = lax.bitcast_convert_type(weight, jnp.int32).reshape(-1)

    h1 = jnp.sum(_hist1(wflat), axis=0)
    c1 = jnp.cumsum(h1)
    excl1 = c1 - h1
    b_hi = jnp.searchsorted(c1, _R_HI, side="right").astype(jnp.int32)
    b_lo = jnp.searchsorted(c1, _R_LO, side="right").astype(jnp.int32)
    r_hi = jnp.int32(_R_HI) - excl1[b_hi]
    r_lo = jnp.int32(_R_LO) - excl1[b_lo]

    targets = jnp.concatenate(
        [jnp.full((16,), b_hi, jnp.int32), jnp.full((16,), b_lo, jnp.int32)]
    )
    h2 = jnp.sum(_hist2(wflat, targets), axis=0)
    ha = h2[:_B2]
    hb = jnp.where(b_hi == b_lo, ha, h2[_B2:])
    m_hi = jnp.searchsorted(jnp.cumsum(ha), r_hi, side="right").astype(jnp.int32)
    m_lo = jnp.searchsorted(jnp.cumsum(hb), r_lo, side="right").astype(jnp.int32)

    q_hi = (b_hi << 19) | (m_hi << 8) | 128
    q_lo = (b_lo << 19) | (m_lo << 8) | 128
    wh = lax.bitcast_convert_type(q_hi, jnp.float32)
    wt = lax.bitcast_convert_type(q_lo, jnp.float32)
    t = 0.5 * (wh + wt)
    t2 = (t * t).reshape(1, 1)

    return _mask(t2, weight)


# speedup vs baseline: 56.6701x; 1.5590x over previous
import functools

import jax
import jax.numpy as jnp
from jax import lax
from jax.experimental import pallas as pl
from jax.experimental.pallas import tpu as pltpu
from jax.experimental.pallas import tpu_sc as plsc

_SPARSITY = 0.5
_TEMP = 1e-05

_N = 4096 * 4096
_LIM = int(min(max(int((1.0 - _SPARSITY) * _N), 0), _N - 2))
_R_HI = _N - 1 - _LIM
_R_LO = _N - 2 - _LIM

_NTILES = 32
_PER_TILE = _N // _NTILES
_CHUNK = 8192
_NCHUNK = _PER_TILE // _CHUNK
_NPAIR = _NCHUNK // 2
_UNROLL = 8
_B1 = 4096
_B2 = 2048
_BW = 4096
_WSHIFT = 10
_SAMP = 16384

_mesh = plsc.VectorSubcoreMesh(core_axis_name="c", subcore_axis_name="s")
_sc_params = pltpu.CompilerParams(needs_layout_passes=False)


def _wid():
    return lax.axis_index("s") * 2 + lax.axis_index("c")


def _zero(hist, nwords):
    zeros = jnp.zeros((16,), jnp.int32)

    def zbody(i, c):
        for j in range(_UNROLL):
            hist[pl.ds(i * 16 * _UNROLL + j * 16, 16)] = zeros
        return c

    lax.fori_loop(0, nwords // (16 * _UNROLL), zbody, 0)


def _reduce_replicas(hist, nb, src_base, src_stride, dst_base):
    def rbody(j, c):
        acc = hist[pl.ds(src_base + j * 16, 16)]
        for k in range(1, 16):
            acc = acc + hist[pl.ds(src_base + k * src_stride + j * 16, 16)]
        hist[pl.ds(dst_base + j * 16, 16)] = acc
        return c

    lax.fori_loop(0, nb // 16, rbody, 0)


def _stream_chunks(w_hbm, base, bufa, bufb, sema, semb, process, carry0):
    def src(c):
        return w_hbm.at[pl.ds(base + c * _CHUNK, _CHUNK)]

    pltpu.async_copy(src(0), bufa, sema)

    def pair_body(p, carry):
        c = 2 * p
        pltpu.async_copy(src(c + 1), bufb, semb)
        pltpu.make_async_copy(src(0), bufa, sema).wait()
        carry = process(bufa, carry)
        nxt = jnp.minimum(c + 2, _NCHUNK - 2)
        pltpu.async_copy(src(nxt), bufa, sema)
        pltpu.make_async_copy(src(0), bufb, semb).wait()
        carry = process(bufb, carry)
        return carry

    carry = lax.fori_loop(0, _NPAIR, pair_body, carry0)
    pltpu.make_async_copy(src(0), bufa, sema).wait()
    return carry


@functools.partial(
    pl.kernel,
    out_type=jax.ShapeDtypeStruct((_NTILES, _B1), jnp.int32),
    mesh=_mesh,
    compiler_params=_sc_params,
    scratch_types=[
        pltpu.VMEM((_SAMP,), jnp.int32),
        pltpu.VMEM((16 * _B1,), jnp.int32),
    ],
)
def _shist(w_hbm, out_hbm, buf, hist):
    wid = _wid()
    base = wid * _PER_TILE
    lane_off = lax.iota(jnp.int32, 16) * _B1
    ones = jnp.ones((16,), jnp.int32)

    _zero(hist, 16 * _B1)
    pltpu.sync_copy(w_hbm.at[pl.ds(base, _SAMP)], buf)

    def vbody(i, cc):
        for j in range(_UNROLL):
            q = buf[pl.ds(i * 16 * _UNROLL + j * 16, 16)] & jnp.int32(0x7FFFFFFF)
            plsc.addupdate_scatter(hist, [lane_off + (q >> 19)], ones)
        return cc

    lax.fori_loop(0, _SAMP // (16 * _UNROLL), vbody, 0)
    _reduce_replicas(hist, _B1, 0, _B1, 0)
    pltpu.sync_copy(hist.at[pl.ds(0, _B1)], out_hbm.at[wid])


@functools.partial(
    pl.kernel,
    out_type=[
        jax.ShapeDtypeStruct((_NTILES, _BW), jnp.int32),
        jax.ShapeDtypeStruct((_NTILES, 16), jnp.int32),
    ],
    mesh=_mesh,
    compiler_params=_sc_params,
    scratch_types=[
        pltpu.VMEM((_CHUNK,), jnp.int32),
        pltpu.VMEM((_CHUNK,), jnp.int32),
        pltpu.VMEM((16,), jnp.int32),
        pltpu.VMEM((16 * _BW,), jnp.int32),
        pltpu.SemaphoreType.DMA,
        pltpu.SemaphoreType.DMA,
    ],
)
def _winpass(w_hbm, qlo_hbm, hist_hbm, below_hbm, bufa, bufb, pvec, hist,
             sema, semb):
    wid = _wid()
    base = wid * _PER_TILE
    lane_off = lax.iota(jnp.int32, 16) * _BW
    ones = jnp.ones((16,), jnp.int32)

    pltpu.sync_copy(qlo_hbm, pvec)
    qlo = pvec[pl.ds(0, 16)]
    _zero(hist, 16 * _BW)

    def process(buf, acc):
        def vbody(i, a):
            for j in range(_UNROLL):
                q = buf[pl.ds(i * 16 * _UNROLL + j * 16, 16)] & jnp.int32(0x7FFFFFFF)
                d = q - qlo
                a = a - (d >> 31)
                in_win = (d >> (_WSHIFT + 12)) == 0
                idx = lane_off + ((d >> _WSHIFT) & (_BW - 1))
                plsc.addupdate_scatter(hist, [idx], ones, mask=in_win)
            return a

        return lax.fori_loop(0, _CHUNK // (16 * _UNROLL), vbody, acc)

    acc = _stream_chunks(
        w_hbm, base, bufa, bufb, sema, semb, process,
        jnp.zeros((16,), jnp.int32),
    )
    pvec[pl.ds(0, 16)] = acc
    pltpu.sync_copy(pvec, below_hbm.at[wid])
    _reduce_replicas(hist, _BW, 0, _BW, 0)
    pltpu.sync_copy(hist.at[pl.ds(0, _BW)], hist_hbm.at[wid])


@functools.partial(
    pl.kernel,
    out_type=jax.ShapeDtypeStruct((_NTILES, _B1), jnp.int32),
    mesh=_mesh,
    compiler_params=_sc_params,
    scratch_types=[
        pltpu.VMEM((_CHUNK,), jnp.int32),
        pltpu.VMEM((_CHUNK,), jnp.int32),
        pltpu.VMEM((16 * _B1,), jnp.int32),
        pltpu.SemaphoreType.DMA,
        pltpu.SemaphoreType.DMA,
    ],
)
def _hist1(w_hbm, out_hbm, bufa, bufb, hist, sema, semb):
    wid = _wid()
    base = wid * _PER_TILE
    lane_off = lax.iota(jnp.int32, 16) * _B1
    ones = jnp.ones((16,), jnp.int32)

    _zero(hist, 16 * _B1)

    def process(buf, carry):
        def vbody(i, cc):
            for j in range(_UNROLL):
                q = buf[pl.ds(i * 16 * _UNROLL + j * 16, 16)] & jnp.int32(0x7FFFFFFF)
                plsc.addupdate_scatter(hist, [lane_off + (q >> 19)], ones)
            return cc

        return lax.fori_loop(0, _CHUNK // (16 * _UNROLL), vbody, carry)

    _stream_chunks(w_hbm, base, bufa, bufb, sema, semb, process, 0)
    _reduce_replicas(hist, _B1, 0, _B1, 0)
    pltpu.sync_copy(hist.at[pl.ds(0, _B1)], out_hbm.at[wid])


@functools.partial(
    pl.kernel,
    out_type=jax.ShapeDtypeStruct((_NTILES, 2 * _B2), jnp.int32),
    mesh=_mesh,
    compiler_params=_sc_params,
    scratch_types=[
        pltpu.VMEM((_CHUNK,), jnp.int32),
        pltpu.VMEM((_CHUNK,), jnp.int32),
        pltpu.VMEM((32,), jnp.int32),
        pltpu.VMEM((32 * _B2,), jnp.int32),
        pltpu.SemaphoreType.DMA,
        pltpu.SemaphoreType.DMA,
    ],
)
def _hist2(w_hbm, targets_hbm, out_hbm, bufa, bufb, tvec, hist, sema, semb):
    wid = _wid()
    base = wid * _PER_TILE
    lane_off = lax.iota(jnp.int32, 16) * _B2
    ones = jnp.ones((16,), jnp.int32)

    pltpu.sync_copy(targets_hbm, tvec)
    pa = tvec[pl.ds(0, 16)]
    pb = tvec[pl.ds(16, 16)]
    b_off = jnp.where(pa != pb, jnp.int32(16 * _B2), jnp.int32(0))

    _zero(hist, 32 * _B2)

    def process(buf, carry):
        def vbody(i, cc):
            for j in range(_UNROLL):
                q = buf[pl.ds(i * 16 * _UNROLL + j * 16, 16)] & jnp.int32(0x7FFFFFFF)
                pfx = q >> 19
                is_b = pfx == pb
                idx = lane_off + ((q >> 8) & (_B2 - 1)) + jnp.where(is_b, b_off, 0)
                plsc.addupdate_scatter(
                    hist, [idx], ones, mask=(pfx == pa) | is_b
                )
            return cc

        return lax.fori_loop(0, _CHUNK // (16 * _UNROLL), vbody, carry)

    _stream_chunks(w_hbm, base, bufa, bufb, sema, semb, process, 0)
    for r in range(2):
        _reduce_replicas(hist, _B2, r * 16 * _B2, _B2, r * _B2)
    pltpu.sync_copy(hist.at[pl.ds(0, 2 * _B2)], out_hbm.at[wid])


def _mask_body(t2_ref, w_ref, o_ref):
    w = w_ref[...]
    d = (t2_ref[0, 0] - w * w) * jnp.float32(1.0 / _TEMP)
    o_ref[...] = w / (1.0 + jnp.exp(d))


_mask = pl.pallas_call(
    _mask_body,
    grid=(16,),
    in_specs=[
        pl.BlockSpec((1, 1), lambda i: (0, 0)),
        pl.BlockSpec((256, 4096), lambda i: (i, 0)),
    ],
    out_specs=pl.BlockSpec((256, 4096), lambda i: (i, 0)),
    out_shape=jax.ShapeDtypeStruct((4096, 4096), jnp.float32),
)


def _exact_t2(wflat):
    h1 = jnp.sum(_hist1(wflat), axis=0)
    c1 = jnp.cumsum(h1)
    excl1 = c1 - h1
    b_hi = jnp.searchsorted(c1, _R_HI, side="right").astype(jnp.int32)
    b_lo = jnp.searchsorted(c1, _R_LO, side="right").astype(jnp.int32)
    r_hi = jnp.int32(_R_HI) - excl1[b_hi]
    r_lo = jnp.int32(_R_LO) - excl1[b_lo]

    targets = jnp.concatenate(
        [jnp.full((16,), b_hi, jnp.int32), jnp.full((16,), b_lo, jnp.int32)]
    )
    h2 = jnp.sum(_hist2(wflat, targets), axis=0)
    ha = h2[:_B2]
    hb = jnp.where(b_hi == b_lo, ha, h2[_B2:])
    m_hi = jnp.searchsorted(jnp.cumsum(ha), r_hi, side="right").astype(jnp.int32)
    m_lo = jnp.searchsorted(jnp.cumsum(hb), r_lo, side="right").astype(jnp.int32)

    q_hi = (b_hi << 19) | (m_hi << 8) | 128
    q_lo = (b_lo << 19) | (m_lo << 8) | 128
    wh = lax.bitcast_convert_type(q_hi, jnp.float32)
    wt = lax.bitcast_convert_type(q_lo, jnp.float32)
    t = 0.5 * (wh + wt)
    return t * t


def kernel(weight):
    wflat = lax.bitcast_convert_type(weight, jnp.int32).reshape(-1)

    hs = jnp.sum(_shist(wflat), axis=0)
    r_s = _R_LO * (_NTILES * _SAMP) // _N
    b_pred = jnp.searchsorted(jnp.cumsum(hs), r_s, side="right").astype(jnp.int32)
    q_lo = jnp.maximum(b_pred - 3, 0) << 19

    hw_parts, below_parts = _winpass(wflat, jnp.full((16,), q_lo, jnp.int32))
    below = jnp.sum(below_parts)
    cumw = below + jnp.cumsum(jnp.sum(hw_parts, axis=0))
    m_hi = jnp.searchsorted(cumw, _R_HI, side="right").astype(jnp.int32)
    m_lo = jnp.searchsorted(cumw, _R_LO, side="right").astype(jnp.int32)
    ok = (jnp.int32(_R_LO) >= below) & (jnp.int32(_R_HI) < cumw[_BW - 1])

    def est_t2(_):
        q_hi_v = q_lo + (m_hi << _WSHIFT) + (1 << (_WSHIFT - 1))
        q_lo_v = q_lo + (m_lo << _WSHIFT) + (1 << (_WSHIFT - 1))
        wh = lax.bitcast_convert_type(q_hi_v, jnp.float32)
        wt = lax.bitcast_convert_type(q_lo_v, jnp.float32)
        t = 0.5 * (wh + wt)
        return t * t

    t2 = lax.cond(ok, est_t2, lambda _: _exact_t2(wflat), operand=None)
    return _mask(t2.reshape(1, 1), weight)


# speedup vs baseline: 102.8976x; 1.8157x over previous
import functools

import jax
import jax.numpy as jnp
from jax import lax
from jax.experimental import pallas as pl
from jax.experimental.pallas import tpu as pltpu
from jax.experimental.pallas import tpu_sc as plsc

_SPARSITY = 0.5
_TEMP = 1e-05

_N = 4096 * 4096
_LIM = int(min(max(int((1.0 - _SPARSITY) * _N), 0), _N - 2))
_R_HI = _N - 1 - _LIM
_R_LO = _N - 2 - _LIM

_NTILES = 32
_PER_TILE = _N // _NTILES
_CHUNK = 8192
_NCHUNK = _PER_TILE // _CHUNK
_NPAIR = _NCHUNK // 2
_UNROLL = 8
_B1 = 4096
_B2 = 2048
_BW = 4096
_WSHIFT = 10
_SAMP = 16384

_mesh = plsc.VectorSubcoreMesh(core_axis_name="c", subcore_axis_name="s")
_sc_params = pltpu.CompilerParams(needs_layout_passes=False)


def _wid():
    return lax.axis_index("s") * 2 + lax.axis_index("c")


def _zero(hist, nwords):
    zeros = jnp.zeros((16,), jnp.int32)

    @plsc.parallel_loop(0, nwords, 16, unroll=8)
    def _(i):
        hist[pl.ds(i, 16)] = zeros


def _reduce_replicas(hist, nb, src_base, src_stride, dst_base):
    @plsc.parallel_loop(0, nb, 16, unroll=4)
    def _(j):
        acc = hist[pl.ds(src_base + j, 16)]
        for k in range(1, 16):
            acc = acc + hist[pl.ds(src_base + k * src_stride + j, 16)]
        hist[pl.ds(dst_base + j, 16)] = acc


def _stream_chunks(w_hbm, base, bufa, bufb, sema, semb, process, carry0):
    def src(c):
        return w_hbm.at[pl.ds(base + c * _CHUNK, _CHUNK)]

    pltpu.async_copy(src(0), bufa, sema)

    def pair_body(p, carry):
        c = 2 * p
        pltpu.async_copy(src(c + 1), bufb, semb)
        pltpu.make_async_copy(src(0), bufa, sema).wait()
        carry = process(bufa, carry)
        nxt = jnp.minimum(c + 2, _NCHUNK - 2)
        pltpu.async_copy(src(nxt), bufa, sema)
        pltpu.make_async_copy(src(0), bufb, semb).wait()
        carry = process(bufb, carry)
        return carry

    carry = lax.fori_loop(0, _NPAIR, pair_body, carry0)
    pltpu.make_async_copy(src(0), bufa, sema).wait()
    return carry


@functools.partial(
    pl.kernel,
    out_type=jax.ShapeDtypeStruct((_NTILES, _B1), jnp.int32),
    mesh=_mesh,
    compiler_params=_sc_params,
    scratch_types=[
        pltpu.VMEM((_SAMP,), jnp.int32),
        pltpu.VMEM((16 * _B1,), jnp.int32),
    ],
)
def _shist(w_hbm, out_hbm, buf, hist):
    wid = _wid()
    base = wid * _PER_TILE
    lane_off = lax.iota(jnp.int32, 16) * _B1
    ones = jnp.ones((16,), jnp.int32)

    _zero(hist, 16 * _B1)
    pltpu.sync_copy(w_hbm.at[pl.ds(base, _SAMP)], buf)

    @plsc.parallel_loop(0, _SAMP, 16, unroll=_UNROLL)
    def _(i):
        q = buf[pl.ds(i, 16)] & jnp.int32(0x7FFFFFFF)
        plsc.addupdate_scatter(hist, [lane_off + (q >> 19)], ones)

    _reduce_replicas(hist, _B1, 0, _B1, 0)
    pltpu.sync_copy(hist.at[pl.ds(0, _B1)], out_hbm.at[wid])


@functools.partial(
    pl.kernel,
    out_type=[
        jax.ShapeDtypeStruct((_NTILES, _BW), jnp.int32),
        jax.ShapeDtypeStruct((_NTILES, 16), jnp.int32),
    ],
    mesh=_mesh,
    compiler_params=_sc_params,
    scratch_types=[
        pltpu.VMEM((_CHUNK,), jnp.int32),
        pltpu.VMEM((_CHUNK,), jnp.int32),
        pltpu.VMEM((16,), jnp.int32),
        pltpu.VMEM((16 * _BW,), jnp.int32),
        pltpu.SemaphoreType.DMA,
        pltpu.SemaphoreType.DMA,
    ],
)
def _winpass(w_hbm, qlo_hbm, hist_hbm, below_hbm, bufa, bufb, pvec, hist,
             sema, semb):
    wid = _wid()
    base = wid * _PER_TILE
    lane_off = lax.iota(jnp.int32, 16) * _BW
    ones = jnp.ones((16,), jnp.int32)

    pltpu.sync_copy(qlo_hbm, pvec)
    qlo = pvec[pl.ds(0, 16)]
    _zero(hist, 16 * _BW)

    def process(buf, acc):
        def vbody(i, a):
            q = buf[pl.ds(i, 16)] & jnp.int32(0x7FFFFFFF)
            d = q - qlo
            a = a - (d >> 31)
            in_win = (d >> (_WSHIFT + 12)) == 0
            idx = lane_off + ((d >> _WSHIFT) & (_BW - 1))
            plsc.addupdate_scatter(hist, [idx], ones, mask=in_win)
            return a

        return plsc.parallel_loop(0, _CHUNK, 16, unroll=_UNROLL, carry=acc)(vbody)

    acc = _stream_chunks(
        w_hbm, base, bufa, bufb, sema, semb, process,
        jnp.zeros((16,), jnp.int32),
    )
    pvec[pl.ds(0, 16)] = acc
    pltpu.sync_copy(pvec, below_hbm.at[wid])
    _reduce_replicas(hist, _BW, 0, _BW, 0)
    pltpu.sync_copy(hist.at[pl.ds(0, _BW)], hist_hbm.at[wid])


@functools.partial(
    pl.kernel,
    out_type=jax.ShapeDtypeStruct((_NTILES, _B1), jnp.int32),
    mesh=_mesh,
    compiler_params=_sc_params,
    scratch_types=[
        pltpu.VMEM((_CHUNK,), jnp.int32),
        pltpu.VMEM((_CHUNK,), jnp.int32),
        pltpu.VMEM((16 * _B1,), jnp.int32),
        pltpu.SemaphoreType.DMA,
        pltpu.SemaphoreType.DMA,
    ],
)
def _hist1(w_hbm, out_hbm, bufa, bufb, hist, sema, semb):
    wid = _wid()
    base = wid * _PER_TILE
    lane_off = lax.iota(jnp.int32, 16) * _B1
    ones = jnp.ones((16,), jnp.int32)

    _zero(hist, 16 * _B1)

    def process(buf, carry):
        @plsc.parallel_loop(0, _CHUNK, 16, unroll=_UNROLL)
        def _(i):
            q = buf[pl.ds(i, 16)] & jnp.int32(0x7FFFFFFF)
            plsc.addupdate_scatter(hist, [lane_off + (q >> 19)], ones)

        return carry

    _stream_chunks(w_hbm, base, bufa, bufb, sema, semb, process, 0)
    _reduce_replicas(hist, _B1, 0, _B1, 0)
    pltpu.sync_copy(hist.at[pl.ds(0, _B1)], out_hbm.at[wid])


@functools.partial(
    pl.kernel,
    out_type=jax.ShapeDtypeStruct((_NTILES, 2 * _B2), jnp.int32),
    mesh=_mesh,
    compiler_params=_sc_params,
    scratch_types=[
        pltpu.VMEM((_CHUNK,), jnp.int32),
        pltpu.VMEM((_CHUNK,), jnp.int32),
        pltpu.VMEM((32,), jnp.int32),
        pltpu.VMEM((32 * _B2,), jnp.int32),
        pltpu.SemaphoreType.DMA,
        pltpu.SemaphoreType.DMA,
    ],
)
def _hist2(w_hbm, targets_hbm, out_hbm, bufa, bufb, tvec, hist, sema, semb):
    wid = _wid()
    base = wid * _PER_TILE
    lane_off = lax.iota(jnp.int32, 16) * _B2
    ones = jnp.ones((16,), jnp.int32)

    pltpu.sync_copy(targets_hbm, tvec)
    pa = tvec[pl.ds(0, 16)]
    pb = tvec[pl.ds(16, 16)]
    b_off = jnp.where(pa != pb, jnp.int32(16 * _B2), jnp.int32(0))

    _zero(hist, 32 * _B2)

    def process(buf, carry):
        @plsc.parallel_loop(0, _CHUNK, 16, unroll=_UNROLL)
        def _(i):
            q = buf[pl.ds(i, 16)] & jnp.int32(0x7FFFFFFF)
            pfx = q >> 19
            is_b = pfx == pb
            idx = lane_off + ((q >> 8) & (_B2 - 1)) + jnp.where(is_b, b_off, 0)
            plsc.addupdate_scatter(hist, [idx], ones, mask=(pfx == pa) | is_b)

        return carry

    _stream_chunks(w_hbm, base, bufa, bufb, sema, semb, process, 0)
    for r in range(2):
        _reduce_replicas(hist, _B2, r * 16 * _B2, _B2, r * _B2)
    pltpu.sync_copy(hist.at[pl.ds(0, 2 * _B2)], out_hbm.at[wid])


def _mask_body(t2_ref, w_ref, o_ref):
    w = w_ref[...]
    d = (w * w - t2_ref[0, 0]) * jnp.float32(0.5 / _TEMP)
    o_ref[...] = w * (0.5 * (1.0 + jnp.tanh(d)))


_mask = pl.pallas_call(
    _mask_body,
    grid=(16,),
    in_specs=[
        pl.BlockSpec((1, 1), lambda i: (0, 0)),
        pl.BlockSpec((256, 4096), lambda i: (i, 0)),
    ],
    out_specs=pl.BlockSpec((256, 4096), lambda i: (i, 0)),
    out_shape=jax.ShapeDtypeStruct((4096, 4096), jnp.float32),
)


def _exact_t2(wflat):
    h1 = jnp.sum(_hist1(wflat), axis=0)
    c1 = jnp.cumsum(h1)
    excl1 = c1 - h1
    b_hi = jnp.searchsorted(c1, _R_HI, side="right").astype(jnp.int32)
    b_lo = jnp.searchsorted(c1, _R_LO, side="right").astype(jnp.int32)
    r_hi = jnp.int32(_R_HI) - excl1[b_hi]
    r_lo = jnp.int32(_R_LO) - excl1[b_lo]

    targets = jnp.concatenate(
        [jnp.full((16,), b_hi, jnp.int32), jnp.full((16,), b_lo, jnp.int32)]
    )
    h2 = jnp.sum(_hist2(wflat, targets), axis=0)
    ha = h2[:_B2]
    hb = jnp.where(b_hi == b_lo, ha, h2[_B2:])
    m_hi = jnp.searchsorted(jnp.cumsum(ha), r_hi, side="right").astype(jnp.int32)
    m_lo = jnp.searchsorted(jnp.cumsum(hb), r_lo, side="right").astype(jnp.int32)

    q_hi = (b_hi << 19) | (m_hi << 8) | 128
    q_lo = (b_lo << 19) | (m_lo << 8) | 128
    wh = lax.bitcast_convert_type(q_hi, jnp.float32)
    wt = lax.bitcast_convert_type(q_lo, jnp.float32)
    t = 0.5 * (wh + wt)
    return t * t


def kernel(weight):
    wflat = lax.bitcast_convert_type(weight, jnp.int32).reshape(-1)

    hs = jnp.sum(_shist(wflat), axis=0)
    r_s = _R_LO * (_NTILES * _SAMP) // _N
    b_pred = jnp.searchsorted(jnp.cumsum(hs), r_s, side="right").astype(jnp.int32)
    q_lo = jnp.maximum(b_pred - 3, 0) << 19

    hw_parts, below_parts = _winpass(wflat, jnp.full((16,), q_lo, jnp.int32))
    below = jnp.sum(below_parts)
    cumw = below + jnp.cumsum(jnp.sum(hw_parts, axis=0))
    m_hi = jnp.searchsorted(cumw, _R_HI, side="right").astype(jnp.int32)
    m_lo = jnp.searchsorted(cumw, _R_LO, side="right").astype(jnp.int32)
    ok = (jnp.int32(_R_LO) >= below) & (jnp.int32(_R_HI) < cumw[_BW - 1])

    def est_t2(_):
        q_hi_v = q_lo + (m_hi << _WSHIFT) + (1 << (_WSHIFT - 1))
        q_lo_v = q_lo + (m_lo << _WSHIFT) + (1 << (_WSHIFT - 1))
        wh = lax.bitcast_convert_type(q_hi_v, jnp.float32)
        wt = lax.bitcast_convert_type(q_lo_v, jnp.float32)
        t = 0.5 * (wh + wt)
        return t * t

    t2 = lax.cond(ok, est_t2, lambda _: _exact_t2(wflat), operand=None)
    return _mask(t2.reshape(1, 1), weight)


# speedup vs baseline: 152.9176x; 1.4861x over previous
import functools

import jax
import jax.numpy as jnp
from jax import lax
from jax.experimental import pallas as pl
from jax.experimental.pallas import tpu as pltpu
from jax.experimental.pallas import tpu_sc as plsc

_SPARSITY = 0.5
_TEMP = 1e-05

_N = 4096 * 4096
_LIM = int(min(max(int((1.0 - _SPARSITY) * _N), 0), _N - 2))
_R_HI = _N - 1 - _LIM
_R_LO = _N - 2 - _LIM

_NTILES = 32
_PER_TILE = _N // _NTILES
_ROWS = 4096
_TROWS = _ROWS // _NTILES
_CHUNK = 8192
_CROWS = 2
_NCHUNK = _PER_TILE // _CHUNK
_NPAIR = _NCHUNK // 2
_UNROLL = 8
_B1 = 4096
_B2 = 2048
_BW = 4096
_WSHIFT = 10
_SAMP = 16384

_mesh = plsc.VectorSubcoreMesh(core_axis_name="c", subcore_axis_name="s")
_sc_params = pltpu.CompilerParams(needs_layout_passes=False)


def _wid():
    return lax.axis_index("s") * 2 + lax.axis_index("c")


def _zero(hist, nwords):
    zeros = jnp.zeros((16,), jnp.int32)

    @plsc.parallel_loop(0, nwords, 16, unroll=8)
    def _(i):
        hist[pl.ds(i, 16)] = zeros


def _reduce_replicas(hist, nb, src_base, src_stride, dst_base):
    @plsc.parallel_loop(0, nb, 16, unroll=4)
    def _(j):
        acc = hist[pl.ds(src_base + j, 16)]
        for k in range(1, 16):
            acc = acc + hist[pl.ds(src_base + k * src_stride + j, 16)]
        hist[pl.ds(dst_base + j, 16)] = acc


def _stream_chunks(w_hbm, row_base, bufa, bufb, sema, semb, process, carry0):
    def src(c):
        return w_hbm.at[pl.ds(row_base + c * _CROWS, _CROWS)]

    pltpu.async_copy(src(0), bufa, sema)

    def pair_body(p, carry):
        c = 2 * p
        pltpu.async_copy(src(c + 1), bufb, semb)
        pltpu.make_async_copy(src(0), bufa, sema).wait()
        carry = process(bufa, carry)
        nxt = jnp.minimum(c + 2, _NCHUNK - 2)
        pltpu.async_copy(src(nxt), bufa, sema)
        pltpu.make_async_copy(src(0), bufb, semb).wait()
        carry = process(bufb, carry)
        return carry

    carry = lax.fori_loop(0, _NPAIR, pair_body, carry0)
    pltpu.make_async_copy(src(0), bufa, sema).wait()
    return carry


@functools.partial(
    pl.kernel,
    out_type=jax.ShapeDtypeStruct((_NTILES, _B1), jnp.int32),
    mesh=_mesh,
    compiler_params=_sc_params,
    scratch_types=[
        pltpu.VMEM((_SAMP // _ROWS, _ROWS), jnp.int32),
        pltpu.VMEM((16 * _B1,), jnp.int32),
    ],
)
def _shist(w_hbm, out_hbm, buf, hist):
    wid = _wid()
    row_base = wid * _TROWS
    lane_off = lax.iota(jnp.int32, 16) * _B1
    ones = jnp.ones((16,), jnp.int32)

    _zero(hist, 16 * _B1)
    pltpu.sync_copy(w_hbm.at[pl.ds(row_base, _SAMP // _ROWS)], buf)

    for r in range(_SAMP // _ROWS):
        @plsc.parallel_loop(0, _ROWS, 16, unroll=_UNROLL)
        def _(i, r=r):
            q = buf[r, pl.ds(i, 16)] & jnp.int32(0x7FFFFFFF)
            plsc.addupdate_scatter(hist, [lane_off + (q >> 19)], ones)

    _reduce_replicas(hist, _B1, 0, _B1, 0)
    pltpu.sync_copy(hist.at[pl.ds(0, _B1)], out_hbm.at[wid])


@functools.partial(
    pl.kernel,
    out_type=[
        jax.ShapeDtypeStruct((_NTILES, _BW), jnp.int32),
        jax.ShapeDtypeStruct((_NTILES, 16), jnp.int32),
    ],
    mesh=_mesh,
    compiler_params=_sc_params,
    scratch_types=[
        pltpu.VMEM((_CROWS, _ROWS), jnp.int32),
        pltpu.VMEM((_CROWS, _ROWS), jnp.int32),
        pltpu.VMEM((16,), jnp.int32),
        pltpu.VMEM((16 * _BW,), jnp.int32),
        pltpu.SemaphoreType.DMA,
        pltpu.SemaphoreType.DMA,
    ],
)
def _winpass(w_hbm, qlo_hbm, hist_hbm, below_hbm, bufa, bufb, pvec, hist,
             sema, semb):
    wid = _wid()
    row_base = wid * _TROWS
    lane_off = lax.iota(jnp.int32, 16) * _BW
    ones = jnp.ones((16,), jnp.int32)

    pltpu.sync_copy(qlo_hbm, pvec)
    qlo = pvec[pl.ds(0, 16)]
    _zero(hist, 16 * _BW)

    def process(buf, acc):
        for r in range(_CROWS):
            def vbody(i, a, r=r):
                q = buf[r, pl.ds(i, 16)] & jnp.int32(0x7FFFFFFF)
                d = q - qlo
                a = a - (d >> 31)
                in_win = (d >> (_WSHIFT + 12)) == 0
                idx = lane_off + ((d >> _WSHIFT) & (_BW - 1))
                plsc.addupdate_scatter(hist, [idx], ones, mask=in_win)
                return a

            acc = plsc.parallel_loop(0, _ROWS, 16, unroll=_UNROLL, carry=acc)(vbody)
        return acc

    acc = _stream_chunks(
        w_hbm, row_base, bufa, bufb, sema, semb, process,
        jnp.zeros((16,), jnp.int32),
    )
    pvec[pl.ds(0, 16)] = acc
    pltpu.sync_copy(pvec, below_hbm.at[wid])
    _reduce_replicas(hist, _BW, 0, _BW, 0)
    pltpu.sync_copy(hist.at[pl.ds(0, _BW)], hist_hbm.at[wid])


@functools.partial(
    pl.kernel,
    out_type=jax.ShapeDtypeStruct((_NTILES, _B1), jnp.int32),
    mesh=_mesh,
    compiler_params=_sc_params,
    scratch_types=[
        pltpu.VMEM((_CROWS, _ROWS), jnp.int32),
        pltpu.VMEM((_CROWS, _ROWS), jnp.int32),
        pltpu.VMEM((16 * _B1,), jnp.int32),
        pltpu.SemaphoreType.DMA,
        pltpu.SemaphoreType.DMA,
    ],
)
def _hist1(w_hbm, out_hbm, bufa, bufb, hist, sema, semb):
    wid = _wid()
    row_base = wid * _TROWS
    lane_off = lax.iota(jnp.int32, 16) * _B1
    ones = jnp.ones((16,), jnp.int32)

    _zero(hist, 16 * _B1)

    def process(buf, carry):
        for r in range(_CROWS):
            @plsc.parallel_loop(0, _ROWS, 16, unroll=_UNROLL)
            def _(i, r=r):
                q = buf[r, pl.ds(i, 16)] & jnp.int32(0x7FFFFFFF)
                plsc.addupdate_scatter(hist, [lane_off + (q >> 19)], ones)

        return carry

    _stream_chunks(w_hbm, row_base, bufa, bufb, sema, semb, process, 0)
    _reduce_replicas(hist, _B1, 0, _B1, 0)
    pltpu.sync_copy(hist.at[pl.ds(0, _B1)], out_hbm.at[wid])


@functools.partial(
    pl.kernel,
    out_type=jax.ShapeDtypeStruct((_NTILES, 2 * _B2), jnp.int32),
    mesh=_mesh,
    compiler_params=_sc_params,
    scratch_types=[
        pltpu.VMEM((_CROWS, _ROWS), jnp.int32),
        pltpu.VMEM((_CROWS, _ROWS), jnp.int32),
        pltpu.VMEM((32,), jnp.int32),
        pltpu.VMEM((32 * _B2,), jnp.int32),
        pltpu.SemaphoreType.DMA,
        pltpu.SemaphoreType.DMA,
    ],
)
def _hist2(w_hbm, targets_hbm, out_hbm, bufa, bufb, tvec, hist, sema, semb):
    wid = _wid()
    row_base = wid * _TROWS
    lane_off = lax.iota(jnp.int32, 16) * _B2
    ones = jnp.ones((16,), jnp.int32)

    pltpu.sync_copy(targets_hbm, tvec)
    pa = tvec[pl.ds(0, 16)]
    pb = tvec[pl.ds(16, 16)]
    b_off = jnp.where(pa != pb, jnp.int32(16 * _B2), jnp.int32(0))

    _zero(hist, 32 * _B2)

    def process(buf, carry):
        for r in range(_CROWS):
            @plsc.parallel_loop(0, _ROWS, 16, unroll=_UNROLL)
            def _(i, r=r):
                q = buf[r, pl.ds(i, 16)] & jnp.int32(0x7FFFFFFF)
                pfx = q >> 19
                is_b = pfx == pb
                idx = lane_off + ((q >> 8) & (_B2 - 1)) + jnp.where(is_b, b_off, 0)
                plsc.addupdate_scatter(hist, [idx], ones, mask=(pfx == pa) | is_b)

        return carry

    _stream_chunks(w_hbm, row_base, bufa, bufb, sema, semb, process, 0)
    for r in range(2):
        _reduce_replicas(hist, _B2, r * 16 * _B2, _B2, r * _B2)
    pltpu.sync_copy(hist.at[pl.ds(0, 2 * _B2)], out_hbm.at[wid])


def _mask_body(t2_ref, w_ref, o_ref):
    w = w_ref[...]
    d = (w * w - t2_ref[0, 0]) * jnp.float32(0.5 / _TEMP)
    o_ref[...] = w * (0.5 * (1.0 + jnp.tanh(d)))


_mask = pl.pallas_call(
    _mask_body,
    grid=(16,),
    in_specs=[
        pl.BlockSpec((1, 1), lambda i: (0, 0)),
        pl.BlockSpec((256, 4096), lambda i: (i, 0)),
    ],
    out_specs=pl.BlockSpec((256, 4096), lambda i: (i, 0)),
    out_shape=jax.ShapeDtypeStruct((4096, 4096), jnp.float32),
)


def _exact_t2(wflat):
    h1 = jnp.sum(_hist1(wflat), axis=0)
    c1 = jnp.cumsum(h1)
    excl1 = c1 - h1
    b_hi = _count_le(c1, _R_HI)
    b_lo = _count_le(c1, _R_LO)
    r_hi = jnp.int32(_R_HI) - excl1[b_hi]
    r_lo = jnp.int32(_R_LO) - excl1[b_lo]

    targets = jnp.concatenate(
        [jnp.full((16,), b_hi, jnp.int32), jnp.full((16,), b_lo, jnp.int32)]
    )
    h2 = jnp.sum(_hist2(wflat, targets), axis=0)
    ha = h2[:_B2]
    hb = jnp.where(b_hi == b_lo, ha, h2[_B2:])
    m_hi = _count_le(jnp.cumsum(ha), r_hi)
    m_lo = _count_le(jnp.cumsum(hb), r_lo)

    q_hi = (b_hi << 19) | (m_hi << 8) | 128
    q_lo = (b_lo << 19) | (m_lo << 8) | 128
    wh = lax.bitcast_convert_type(q_hi, jnp.float32)
    wt = lax.bitcast_convert_type(q_lo, jnp.float32)
    t = 0.5 * (wh + wt)
    return t * t


def _count_le(cum, r):
    return jnp.sum((cum <= r).astype(jnp.int32)).astype(jnp.int32)


def kernel(weight):
    wflat = lax.bitcast_convert_type(weight, jnp.int32)

    hs = jnp.sum(_shist(wflat), axis=0)
    r_s = _R_LO * (_NTILES * _SAMP) // _N
    b_pred = _count_le(jnp.cumsum(hs), r_s)
    q_lo = jnp.maximum(b_pred - 3, 0) << 19

    hw_parts, below_parts = _winpass(wflat, jnp.full((16,), q_lo, jnp.int32))
    below = jnp.sum(below_parts)
    cumw = below + jnp.cumsum(jnp.sum(hw_parts, axis=0))
    m_hi = _count_le(cumw, _R_HI)
    m_lo = _count_le(cumw, _R_LO)
    ok = (jnp.int32(_R_LO) >= below) & (jnp.int32(_R_HI) < cumw[_BW - 1])

    def est_t2(_):
        q_hi_v = q_lo + (m_hi << _WSHIFT) + (1 << (_WSHIFT - 1))
        q_lo_v = q_lo + (m_lo << _WSHIFT) + (1 << (_WSHIFT - 1))
        wh = lax.bitcast_convert_type(q_hi_v, jnp.float32)
        wt = lax.bitcast_convert_type(q_lo_v, jnp.float32)
        t = 0.5 * (wh + wt)
        return t * t

    t2 = lax.cond(ok, est_t2, lambda _: _exact_t2(wflat), operand=None)
    return _mask(t2.reshape(1, 1), weight)
